# speedup vs baseline: 1.0095x; 1.0095x over previous
import functools

import jax
import jax.numpy as jnp
from jax.experimental import pallas as pl

N_NODES = 10000
N_EDGES_TOT = 320000 + 10000
HEADS = 8
HID = 64
NEG = 0.2
SLOPE = 0.005


def _mm_body(x_ref, w_ref, o_ref):
    o_ref[...] = jnp.dot(x_ref[...], w_ref[...],
                         preferred_element_type=jnp.float32)


def _matmul(x, w, block_rows=1000):
    n, k = x.shape
    _, m = w.shape
    grid = (n // block_rows,)
    return pl.pallas_call(
        _mm_body,
        grid=grid,
        in_specs=[
            pl.BlockSpec((block_rows, k), lambda i: (i, 0)),
            pl.BlockSpec((k, m), lambda i: (0, 0)),
        ],
        out_specs=pl.BlockSpec((block_rows, m), lambda i: (i, 0)),
        out_shape=jax.ShapeDtypeStruct((n, m), jnp.float32),
    )(x, w)


def _gat_conv(x, src, dst, W, a_src, a_dst, b, H, C):
    N = x.shape[0]
    xw = _matmul(x, W).reshape(N, H, C)
    al_s = (xw * a_src).sum(-1)
    al_d = (xw * a_dst).sum(-1)
    alpha = jax.nn.leaky_relu(al_s[src] + al_d[dst], NEG)
    amax = jax.ops.segment_max(alpha, dst, num_segments=N)
    e = jnp.exp(alpha - amax[dst])
    denom = jax.ops.segment_sum(e, dst, num_segments=N)
    coef = e / (denom[dst] + 1e-16)
    out = jax.ops.segment_sum(xw[src] * coef[:, :, None], dst, num_segments=N)
    return out.reshape(N, H * C) + b


def kernel(x, W0, as0, ad0, b0, W1, as1, ad1, b1, W2, as2, ad2, b2,
           W3, as3, ad3, b3, W4, as4, ad4, b4, W5, as5, ad5, b5,
           Wfc, bfc, edge_index):
    loop = jnp.arange(N_NODES, dtype=edge_index.dtype)
    src = jnp.concatenate([edge_index[0], loop])
    dst = jnp.concatenate([edge_index[1], loop])
    layers = [(W0, as0, ad0, b0), (W1, as1, ad1, b1), (W2, as2, ad2, b2),
              (W3, as3, ad3, b3), (W4, as4, ad4, b4)]
    h = x
    for (W, a_s, a_d, b) in layers:
        h = _gat_conv(h, src, dst, W, a_s, a_d, b, HEADS, HID)
        h = jax.nn.leaky_relu(h, SLOPE)
    h = _gat_conv(h, src, dst, W5, as5, ad5, b5, 1, 1)
    h = jax.nn.leaky_relu(h, SLOPE).reshape(1, N_NODES)
    return h @ Wfc.T + bfc
